# baseline (speedup 1.0000x reference)
import functools

import jax
import jax.numpy as jnp
from jax import lax
from jax.experimental import pallas as pl
from jax.experimental.pallas import tpu as pltpu
from jax.experimental.pallas import tpu_sc as plsc

N = 100000
C = 20
S = 1000
ALPHA = 0.5

BR = 2048
GRID_A = 32
GRID_B = 12
GRID_C = 5
SPLIT_A = GRID_A * BR
SPLIT_B = GRID_B * BR
BASE_C = SPLIT_A + SPLIT_B

SEGP = 1024
NW = 32
NEG = -3.0e38


def _ce_body(x_ref, t_ref, o_ref):
    x = x_ref[...]
    se = jnp.sum(jnp.exp(x), axis=0)
    t = t_ref[...].reshape(1, BR)
    sub = lax.broadcasted_iota(jnp.int32, (C, BR), 0)
    tl = jnp.sum(jnp.where(sub == t, x, 0.0), axis=0)
    yi = se.view(jnp.int32)
    ef = ((yi >> 23) - 127).astype(jnp.float32)
    m = ((yi & 0x7FFFFF) | 0x3F800000).view(jnp.float32)
    pz = jnp.float32(-0.054862853)
    pz = pz * m + 0.43586185
    pz = pz * m - 1.4424810
    pz = pz * m + 2.7922552
    pz = pz * m - 1.7306317
    loss = 0.6931472 * ef + pz - tl
    o_ref[...] = loss.reshape(16, 128)


def _make_ce(grid, blk0, out_rows):
    return pl.pallas_call(
        _ce_body,
        grid=(grid,),
        in_specs=[
            pl.BlockSpec((C, BR), lambda i: (0, i + blk0)),
            pl.BlockSpec((BR,), lambda i: (i + blk0,)),
        ],
        out_specs=pl.BlockSpec((16, 128), lambda i: (i, 0)),
        out_shape=jax.ShapeDtypeStruct((out_rows, 128), jnp.float32),
    )


_tc_ce_a = _make_ce(GRID_A, 0, 512)
_tc_ce_b = _make_ce(GRID_B, GRID_A, 200)
_tc_ce_c = _make_ce(GRID_C, GRID_A + GRID_B, 88)


_MESH = plsc.VectorSubcoreMesh(core_axis_name="c", subcore_axis_name="s")


def _make_sc_seg(row_base, chunk, valid_rows):
    ng = chunk // 16
    last_rows = valid_rows - (NW - 1) * chunk
    ng_last = last_rows // 16
    uniform = last_rows == chunk

    def body(loss_hbm, cid_hbm, out_sum, out_max, lv, cid_v, ssum, smax):
        cid_core = lax.axis_index("c")
        sid = lax.axis_index("s")
        wid = sid * 2 + cid_core
        is_last = wid == NW - 1
        not_last = jnp.logical_not(is_last)
        rows0 = wid * chunk

        iota = lax.broadcasted_iota(jnp.int32, (16,), 0)
        zeros16 = jnp.zeros((16,), jnp.float32)
        negs16 = jnp.full((16,), NEG, jnp.float32)

        def init_body(i, carry):
            ssum[pl.ds(i * 16, 16)] = zeros16
            smax[pl.ds(i * 16, 16)] = negs16
            return carry

        lax.fori_loop(0, SEGP // 16, init_body, 0)

        asta = (rows0 >> 7) // 8 * 8
        delta = rows0 - asta * 128
        pltpu.sync_copy(loss_hbm.at[pl.ds(asta, 16), :], lv)

        if uniform:
            pltpu.sync_copy(
                cid_hbm.at[pl.ds(row_base + rows0, chunk)], cid_v)
        else:
            @pl.when(not_last)
            def _():
                pltpu.sync_copy(
                    cid_hbm.at[pl.ds(row_base + rows0, chunk)], cid_v)

            @pl.when(is_last)
            def _():
                pltpu.sync_copy(
                    cid_hbm.at[pl.ds(row_base + (NW - 1) * chunk, last_rows)],
                    cid_v.at[pl.ds(0, last_rows)])

        _dnums = lax.GatherDimensionNumbers(
            offset_dims=(), collapsed_slice_dims=(0,), start_index_map=(0,))

        def lane_take(x, idx):
            return lax.gather(x, idx[:, None], _dnums, (1,),
                              mode=lax.GatherScatterMode.PROMISE_IN_BOUNDS)

        def group_body(g, carry):
            off = delta + g * 16 + iota
            loss = plsc.load_gather(lv, [off >> 7, off & 127])
            ids = cid_v[pl.ds(g * 16, 16)]
            plsc.addupdate_scatter(ssum, [ids], loss)
            rm = loss
            for d in (1, 2, 4, 8):
                idx = jnp.maximum(iota - d, 0)
                same = jnp.logical_and(
                    lane_take(ids, idx) == ids, iota >= d)
                rm = jnp.maximum(
                    rm, jnp.where(same, lane_take(rm, idx), NEG))
            nxt = jnp.minimum(iota + 1, 15)
            lastm = jnp.logical_or(lane_take(ids, nxt) != ids, iota == 15)
            cm = plsc.load_gather(smax, [ids])
            plsc.store_scatter(smax, [ids], jnp.maximum(cm, rm), mask=lastm)
            return carry

        if uniform:
            lax.fori_loop(0, ng, group_body, 0)
        else:
            @pl.when(not_last)
            def _():
                lax.fori_loop(0, ng, group_body, 0)

            @pl.when(is_last)
            def _():
                lax.fori_loop(0, ng_last, group_body, 0)

        pltpu.sync_copy(ssum, out_sum.at[wid])
        pltpu.sync_copy(smax, out_max.at[wid])

    return functools.partial(
        pl.kernel,
        out_type=(jax.ShapeDtypeStruct((NW, SEGP), jnp.float32),
                  jax.ShapeDtypeStruct((NW, SEGP), jnp.float32)),
        mesh=_MESH,
        compiler_params=pltpu.CompilerParams(needs_layout_passes=False),
        scratch_types=[
            pltpu.VMEM((16, 128), jnp.float32),
            pltpu.VMEM((chunk,), jnp.int32),
            pltpu.VMEM((SEGP,), jnp.float32),
            pltpu.VMEM((SEGP,), jnp.float32),
        ],
    )(body)


_sc_seg_a = _make_sc_seg(0, 2048, SPLIT_A)
_sc_seg_b = _make_sc_seg(SPLIT_A, 768, SPLIT_B)
_sc_seg_c = _make_sc_seg(BASE_C, 304, N - BASE_C)


def _tc_body(sa_ref, ma_ref, sb_ref, mb_ref, sc_ref, mc_ref, o_ref):
    s = (jnp.sum(sa_ref[...], axis=0) + jnp.sum(sb_ref[...], axis=0)
         + jnp.sum(sc_ref[...], axis=0))
    m = jnp.maximum(
        jnp.maximum(jnp.max(ma_ref[...], axis=0), jnp.max(mb_ref[...], axis=0)),
        jnp.max(mc_ref[...], axis=0))
    ci = jnp.max(lax.broadcasted_iota(jnp.int32, (NW, SEGP), 1), axis=0)
    msk = jnp.logical_and(m > -1.0e30, ci < S)
    comb = ALPHA * s + (1.0 - ALPHA) * m
    total = jnp.sum(jnp.where(msk, comb, 0.0))
    n = jnp.maximum(jnp.sum(msk.astype(jnp.float32)), 1.0)
    o_ref[0, 0] = total / n


_tc_combine = pl.pallas_call(
    _tc_body,
    out_shape=jax.ShapeDtypeStruct((1, 1), jnp.float32),
    out_specs=pl.BlockSpec(memory_space=pltpu.SMEM),
)


def kernel(logits, targets, complex_id):
    lt = logits.T
    la = _tc_ce_a(lt, targets)
    sa, ma = _sc_seg_a(la, complex_id)
    lb = _tc_ce_b(lt, targets)
    sb, mb = _sc_seg_b(lb, complex_id)
    lc = _tc_ce_c(lt, targets)
    sc, mc = _sc_seg_c(lc, complex_id)
    out = _tc_combine(sa, ma, sb, mb, sc, mc)
    return out[0, 0]


# speedup vs baseline: 1.0639x; 1.0639x over previous
import functools

import jax
import jax.numpy as jnp
from jax import lax
from jax.experimental import pallas as pl
from jax.experimental.pallas import tpu as pltpu
from jax.experimental.pallas import tpu_sc as plsc

N = 100000
C = 20
S = 1000
ALPHA = 0.5

BR = 2048
GRID_A = 28
GRID_B = 21
SPLIT_A = GRID_A * BR

SEGP = 1024
NW = 32
NEG = -3.0e38


def _ce_body(x_ref, t_ref, o_ref):
    x = x_ref[...]
    se = jnp.sum(jnp.exp(x), axis=0)
    t = t_ref[...].reshape(1, BR)
    sub = lax.broadcasted_iota(jnp.int32, (C, BR), 0)
    tl = jnp.sum(jnp.where(sub == t, x, 0.0), axis=0)
    yi = se.view(jnp.int32)
    ef = ((yi >> 23) - 127).astype(jnp.float32)
    m = ((yi & 0x7FFFFF) | 0x3F800000).view(jnp.float32)
    pz = jnp.float32(-0.054862853)
    pz = pz * m + 0.43586185
    pz = pz * m - 1.4424810
    pz = pz * m + 2.7922552
    pz = pz * m - 1.7306317
    loss = 0.6931472 * ef + pz - tl
    o_ref[...] = loss.reshape(16, 128)


def _make_ce(grid, blk0, out_rows):
    return pl.pallas_call(
        _ce_body,
        grid=(grid,),
        in_specs=[
            pl.BlockSpec((C, BR), lambda i: (0, i + blk0)),
            pl.BlockSpec((BR,), lambda i: (i + blk0,)),
        ],
        out_specs=pl.BlockSpec((16, 128), lambda i: (i, 0)),
        out_shape=jax.ShapeDtypeStruct((out_rows, 128), jnp.float32),
    )


_tc_ce_a = _make_ce(GRID_A, 0, 456)
_tc_ce_b = _make_ce(GRID_B, GRID_A, 344)


_MESH = plsc.VectorSubcoreMesh(core_axis_name="c", subcore_axis_name="s")


def _make_sc_seg(row_base, chunk, valid_rows):
    ng = chunk // 16
    last_rows = valid_rows - (NW - 1) * chunk
    ng_last = last_rows // 16
    uniform = last_rows == chunk

    def body(loss_hbm, cid_hbm, out_sum, out_max, lv, cid_v, ssum, smax):
        cid_core = lax.axis_index("c")
        sid = lax.axis_index("s")
        wid = sid * 2 + cid_core
        is_last = wid == NW - 1
        not_last = jnp.logical_not(is_last)
        rows0 = wid * chunk

        iota = lax.broadcasted_iota(jnp.int32, (16,), 0)
        zeros16 = jnp.zeros((16,), jnp.float32)
        negs16 = jnp.full((16,), NEG, jnp.float32)

        def init_body(i, carry):
            ssum[pl.ds(i * 16, 16)] = zeros16
            smax[pl.ds(i * 16, 16)] = negs16
            return carry

        lax.fori_loop(0, SEGP // 16, init_body, 0)

        asta = (rows0 >> 7) // 8 * 8
        delta = rows0 - asta * 128
        pltpu.sync_copy(loss_hbm.at[pl.ds(asta, 24), :], lv)

        if uniform:
            pltpu.sync_copy(
                cid_hbm.at[pl.ds(row_base + rows0, chunk)], cid_v)
        else:
            @pl.when(not_last)
            def _():
                pltpu.sync_copy(
                    cid_hbm.at[pl.ds(row_base + rows0, chunk)], cid_v)

            @pl.when(is_last)
            def _():
                pltpu.sync_copy(
                    cid_hbm.at[pl.ds(row_base + (NW - 1) * chunk, last_rows)],
                    cid_v.at[pl.ds(0, last_rows)])

        _dnums = lax.GatherDimensionNumbers(
            offset_dims=(), collapsed_slice_dims=(0,), start_index_map=(0,))

        def lane_take(x, idx):
            return lax.gather(x, idx[:, None], _dnums, (1,),
                              mode=lax.GatherScatterMode.PROMISE_IN_BOUNDS)

        def group_body(g, carry):
            off = delta + g * 16 + iota
            loss = plsc.load_gather(lv, [off >> 7, off & 127])
            ids = cid_v[pl.ds(g * 16, 16)]
            plsc.addupdate_scatter(ssum, [ids], loss)
            rm = loss
            for d in (1, 2, 4, 8):
                idx = jnp.maximum(iota - d, 0)
                same = jnp.logical_and(
                    lane_take(ids, idx) == ids, iota >= d)
                rm = jnp.maximum(
                    rm, jnp.where(same, lane_take(rm, idx), NEG))
            nxt = jnp.minimum(iota + 1, 15)
            lastm = jnp.logical_or(lane_take(ids, nxt) != ids, iota == 15)
            cm = plsc.load_gather(smax, [ids])
            plsc.store_scatter(smax, [ids], jnp.maximum(cm, rm), mask=lastm)
            return carry

        if uniform:
            lax.fori_loop(0, ng, group_body, 0)
        else:
            @pl.when(not_last)
            def _():
                lax.fori_loop(0, ng, group_body, 0)

            @pl.when(is_last)
            def _():
                lax.fori_loop(0, ng_last, group_body, 0)

        pltpu.sync_copy(ssum, out_sum.at[wid])
        pltpu.sync_copy(smax, out_max.at[wid])

    return functools.partial(
        pl.kernel,
        out_type=(jax.ShapeDtypeStruct((NW, SEGP), jnp.float32),
                  jax.ShapeDtypeStruct((NW, SEGP), jnp.float32)),
        mesh=_MESH,
        compiler_params=pltpu.CompilerParams(needs_layout_passes=False),
        scratch_types=[
            pltpu.VMEM((24, 128), jnp.float32),
            pltpu.VMEM((chunk,), jnp.int32),
            pltpu.VMEM((SEGP,), jnp.float32),
            pltpu.VMEM((SEGP,), jnp.float32),
        ],
    )(body)


_sc_seg_a = _make_sc_seg(0, 1792, SPLIT_A)
_sc_seg_b = _make_sc_seg(SPLIT_A, 1344, N - SPLIT_A)


def _tc_body(sa_ref, ma_ref, sb_ref, mb_ref, o_ref):
    s = jnp.sum(sa_ref[...], axis=0) + jnp.sum(sb_ref[...], axis=0)
    m = jnp.maximum(jnp.max(ma_ref[...], axis=0), jnp.max(mb_ref[...], axis=0))
    ci = jnp.max(lax.broadcasted_iota(jnp.int32, (NW, SEGP), 1), axis=0)
    msk = jnp.logical_and(m > -1.0e30, ci < S)
    comb = ALPHA * s + (1.0 - ALPHA) * m
    total = jnp.sum(jnp.where(msk, comb, 0.0))
    n = jnp.maximum(jnp.sum(msk.astype(jnp.float32)), 1.0)
    o_ref[0, 0] = total / n


_tc_combine = pl.pallas_call(
    _tc_body,
    out_shape=jax.ShapeDtypeStruct((1, 1), jnp.float32),
    out_specs=pl.BlockSpec(memory_space=pltpu.SMEM),
)


def kernel(logits, targets, complex_id):
    lt = logits.T
    la = _tc_ce_a(lt, targets)
    sa, ma = _sc_seg_a(la, complex_id)
    lb = _tc_ce_b(lt, targets)
    sb, mb = _sc_seg_b(lb, complex_id)
    out = _tc_combine(sa, ma, sb, mb)
    return out[0, 0]
